# baseline (speedup 1.0000x reference)
import functools

import jax
import jax.numpy as jnp
from jax import lax
from jax.experimental import pallas as pl
from jax.experimental.pallas import tpu as pltpu
from jax.experimental.pallas import tpu_sc as plsc

N = 10000
HID = 256
HEADS = 4
NC = 2
NS = 16
NW = NC * NS
NPW = 320
N_PAD = NW * NPW
E = 320000 + N
E_PW = 11264
K = 32
NCK = E_PW // K
SROWS = (NPW + 2) * 4

_CP = pltpu.CompilerParams(needs_layout_passes=False)


def _mm(a, b):
    M, Kd = a.shape
    Nd = b.shape[1]
    BM = 512
    assert M % BM == 0

    def mmk(a_ref, b_ref, o_ref):
        o_ref[...] = jnp.dot(a_ref[...], b_ref[...],
                             preferred_element_type=jnp.float32)

    return pl.pallas_call(
        mmk,
        grid=(M // BM,),
        in_specs=[pl.BlockSpec((BM, Kd), lambda i: (i, 0)),
                  pl.BlockSpec((Kd, Nd), lambda i: (0, 0))],
        out_specs=pl.BlockSpec((BM, Nd), lambda i: (i, 0)),
        out_shape=jax.ShapeDtypeStruct((M, Nd), jnp.float32),
    )(a, b)


def _mesh():
    return plsc.VectorSubcoreMesh(core_axis_name="c", subcore_axis_name="s")


def _attn_kernel(srcs_p, dsts_p, asrc128, adst128, btile):
    @functools.partial(
        pl.kernel,
        out_type=jax.ShapeDtypeStruct((NW, E_PW * 4), jnp.float32),
        mesh=_mesh(),
        compiler_params=_CP,
        scratch_types=[
            pltpu.VMEM((E_PW,), jnp.int32),
            pltpu.VMEM((E_PW,), jnp.int32),
            pltpu.VMEM((SROWS,), jnp.float32),
            pltpu.VMEM((2, K, 128), jnp.float32),
            pltpu.VMEM((2, K, 128), jnp.float32),
            pltpu.VMEM((E_PW * 4,), jnp.float32),
            pltpu.VMEM((16,), jnp.float32),
            pltpu.SemaphoreType.DMA((2, 2)),
        ],
    )
    def body(srcs_hbm, dsts_hbm, asrc_hbm, adst_hbm, b_hbm, w_hbm,
             srcs_w, dsts_w, s_ref, ar_ring, ad_ring, pbuf, bvec, sems):
        wid = lax.axis_index("s") * NC + lax.axis_index("c")
        node0 = wid * NPW
        iota = lax.iota(jnp.int32, 16)
        lane_h = iota & 3
        lane_e = iota >> 2
        z16 = jnp.zeros((16,), jnp.float32)
        zi16 = jnp.zeros((16,), jnp.int32)

        pltpu.sync_copy(srcs_hbm.at[wid], srcs_w)
        pltpu.sync_copy(dsts_hbm.at[wid], dsts_w)
        pltpu.sync_copy(b_hbm, bvec)
        b = bvec[...]

        def zs(i, c):
            s_ref[pl.ds(i * 16, 16)] = z16
            return c
        lax.fori_loop(0, SROWS // 16, zs, 0)

        def start(ck, slot):
            pltpu.async_copy(asrc_hbm.at[srcs_w.at[pl.ds(ck * K, K)]],
                             ar_ring.at[slot], sems.at[0, slot])
            pltpu.async_copy(adst_hbm.at[dsts_w.at[pl.ds(ck * K, K)]],
                             ad_ring.at[slot], sems.at[1, slot])

        def wait(ck, slot):
            pltpu.make_async_copy(asrc_hbm.at[srcs_w.at[pl.ds(ck * K, K)]],
                                  ar_ring.at[slot], sems.at[0, slot]).wait()
            pltpu.make_async_copy(adst_hbm.at[dsts_w.at[pl.ds(ck * K, K)]],
                                  ad_ring.at[slot], sems.at[1, slot]).wait()

        start(0, 0)

        def p1(ck, c):
            slot = ck & 1

            @pl.when(ck + 1 < NCK)
            def _():
                start(ck + 1, 1 - slot)
            wait(ck, slot)
            sl = zi16 + slot
            for j in range(K // 4):
                le = 4 * j + lane_e
                av = plsc.load_gather(ar_ring, [sl, le, lane_h])
                dv = plsc.load_gather(ad_ring, [sl, le, lane_h])
                alpha = av + dv
                alpha = jnp.maximum(alpha, 0.2 * alpha)
                p = jnp.exp(alpha - b)
                pbuf[pl.ds(ck * (K * 4) + 16 * j, 16)] = p
                dstv = plsc.load_gather(dsts_w, [ck * K + le])
                sidx = (dstv - node0) * 4 + lane_h
                for k in range(4):
                    plsc.addupdate_scatter(s_ref, [sidx], p,
                                           mask=lane_e == k)
            return c
        lax.fori_loop(0, NCK, p1, 0)

        def rec(i, c):
            v = s_ref[pl.ds(i * 16, 16)]
            s_ref[pl.ds(i * 16, 16)] = 1.0 / (v + 1e-16)
            return c
        lax.fori_loop(0, SROWS // 16, rec, 0)

        def p2(ck, c):
            for j in range(K // 4):
                off = ck * (K * 4) + 16 * j
                le = 4 * j + lane_e
                dstv = plsc.load_gather(dsts_w, [ck * K + le])
                r = plsc.load_gather(s_ref, [(dstv - node0) * 4 + lane_h])
                pbuf[pl.ds(off, 16)] = pbuf[pl.ds(off, 16)] * r
            return c
        lax.fori_loop(0, NCK, p2, 0)
        pltpu.sync_copy(pbuf, w_hbm.at[wid])

    return body(srcs_p, dsts_p, asrc128, adst128, btile)


def _agg_kernel(srcs_p, dsts_p, w_p, h_pad, zg):
    @functools.partial(
        pl.kernel,
        out_type=jax.ShapeDtypeStruct((4, N_PAD, HID), jnp.float32),
        mesh=_mesh(),
        compiler_params=_CP,
        scratch_types=[
            pltpu.VMEM((E_PW,), jnp.int32),
            pltpu.VMEM((E_PW,), jnp.int32),
            pltpu.VMEM((2, K, HID), jnp.float32),
            pltpu.VMEM((2, K * 4), jnp.float32),
            pltpu.VMEM((NPW + 1, HID), jnp.float32),
            pltpu.SemaphoreType.DMA((2, 2)),
        ],
    )
    def body(srcs_hbm, dsts_hbm, w_hbm, h_hbm, zg_hbm, g4_hbm,
             srcs_w, dsts_w, h_ring, w_ring, gstage, sems):
        wid = lax.axis_index("s") * NC + lax.axis_index("c")
        node0 = wid * NPW
        iota = lax.iota(jnp.int32, 16)
        zi16 = jnp.zeros((16,), jnp.int32)

        pltpu.sync_copy(srcs_hbm.at[wid], srcs_w)
        pltpu.sync_copy(dsts_hbm.at[wid], dsts_w)

        def start(ck, slot):
            pltpu.async_copy(h_hbm.at[srcs_w.at[pl.ds(ck * K, K)]],
                             h_ring.at[slot], sems.at[0, slot])
            pltpu.async_copy(w_hbm.at[wid, pl.ds(ck * (K * 4), K * 4)],
                             w_ring.at[slot], sems.at[1, slot])

        def wait(ck, slot):
            pltpu.make_async_copy(h_hbm.at[srcs_w.at[pl.ds(ck * K, K)]],
                                  h_ring.at[slot], sems.at[0, slot]).wait()
            pltpu.make_async_copy(w_hbm.at[wid, pl.ds(ck * (K * 4), K * 4)],
                                  w_ring.at[slot], sems.at[1, slot]).wait()

        def q_body(q, c):
            pltpu.sync_copy(zg_hbm, gstage)
            start(0, 0)

            def p2(ck, c2):
                slot = ck & 1

                @pl.when(ck + 1 < NCK)
                def _():
                    start(ck + 1, 1 - slot)
                wait(ck, slot)
                sl = zi16 + slot
                for e in range(K):
                    dstv = plsc.load_gather(dsts_w, [zi16 + (ck * K + e)])
                    dstl = dstv - node0
                    for h in range(HEADS):
                        wh = plsc.load_gather(w_ring, [sl, zi16 + (e * 4 + h)])
                        for i in range(4):
                            hq = plsc.load_gather(
                                h_ring, [sl, zi16 + e, q * 64 + i * 16 + iota])
                            plsc.addupdate_scatter(
                                gstage, [dstl, h * 64 + i * 16 + iota], wh * hq)
                return c2
            lax.fori_loop(0, NCK, p2, 0)
            pltpu.sync_copy(gstage.at[pl.ds(0, NPW)],
                            g4_hbm.at[q, pl.ds(node0, NPW)])
            return c
        lax.fori_loop(0, 4, q_body, 0)

    return body(srcs_p, dsts_p, w_p, h_pad, zg)


def kernel(x, edge_index, batch, t, cond, node_W, node_b, time_W1, time_b1,
           time_W2, time_b2, cond_W1, cond_b1, cond_W2, cond_b2,
           gat_W0, att_src0, att_dst0, gat_b0,
           gat_W1, att_src1, att_dst1, gat_b1,
           gat_W2, att_src2, att_dst2, gat_b2, out_W, out_b):
    n = x.shape[0]
    t_emb = jax.nn.relu(t[:, None] @ time_W1 + time_b1) @ time_W2 + time_b2
    c_emb = jax.nn.relu(cond @ cond_W1 + cond_b1) @ cond_W2 + cond_b2
    xp = jnp.pad(x, ((0, N_PAD - n), (0, 0)))
    emb = jnp.pad((t_emb + c_emb)[batch], ((0, N_PAD - n), (0, 0)))
    hp = _mm(xp, node_W) + node_b + emb

    loop = jnp.arange(n, dtype=edge_index.dtype)
    src = jnp.concatenate([edge_index[0], loop])
    dst = jnp.concatenate([edge_index[1], loop])
    order = jnp.argsort(dst)
    srcs, dsts = src[order], dst[order]
    bounds = jnp.arange(NW + 1, dtype=jnp.int32) * NPW
    estart = jnp.searchsorted(dsts, bounds).astype(jnp.int32)
    i = jnp.arange(E_PW, dtype=jnp.int32)[None, :]
    ne = (estart[1:] - estart[:-1])[:, None]
    idx = jnp.minimum(estart[:-1][:, None] + i, E - 1)
    valid = i < ne
    srcs_p = jnp.where(valid, srcs[idx], 0)
    dsts_p = jnp.where(valid, dsts[idx],
                       (jnp.arange(NW, dtype=jnp.int32) * NPW + NPW)[:, None])

    zg = jnp.zeros((NPW + 1, HID), jnp.float32)
    for (Wl, a_s, a_d, bl) in ((gat_W0, att_src0, att_dst0, gat_b0),
                               (gat_W1, att_src1, att_dst1, gat_b1),
                               (gat_W2, att_src2, att_dst2, gat_b2)):
        Wr = Wl.reshape(HID, HEADS, HID)
        A_src = jnp.einsum('chd,hd->ch', Wr, a_s)
        A_dst = jnp.einsum('chd,hd->ch', Wr, a_d)
        asrc128 = _mm(hp, jnp.pad(A_src, ((0, 0), (0, 124))))
        adst128 = jnp.pad(_mm(hp, jnp.pad(A_dst, ((0, 0), (0, 124)))),
                          ((0, 8), (0, 0)))
        b4 = jax.nn.leaky_relu(asrc128[:n, :4].max(0)
                               + adst128[:n, :4].max(0), 0.2)
        btile = jnp.tile(b4, 4)
        w_p = _attn_kernel(srcs_p, dsts_p, asrc128, adst128, btile)
        g4 = _agg_kernel(srcs_p, dsts_p, w_p, hp, zg)
        G2 = (g4.reshape(4, N_PAD, HEADS, 64)
              .transpose(1, 2, 0, 3).reshape(N_PAD, HEADS * HID))
        Wcat = Wr.transpose(1, 0, 2).reshape(HEADS * HID, HID)
        hp = jax.nn.relu(_mm(G2, Wcat) / HEADS + bl)
    return _mm(hp, out_W)[:n] + out_b


# speedup vs baseline: 1.1793x; 1.1793x over previous
import functools

import jax
import jax.numpy as jnp
from jax import lax
from jax.experimental import pallas as pl
from jax.experimental.pallas import tpu as pltpu
from jax.experimental.pallas import tpu_sc as plsc

N = 10000
HID = 256
HEADS = 4
NC = 2
NS = 16
NW = NC * NS
NPW = 320
N_PAD = NW * NPW
E = 320000 + N
E_PW = 11264
K = 32
NCK = E_PW // K
SROWS = (NPW + 2) * 4

_CP = pltpu.CompilerParams(needs_layout_passes=False)


def _mm(a, b):
    M, Kd = a.shape
    Nd = b.shape[1]
    BM = 512
    assert M % BM == 0

    def mmk(a_ref, b_ref, o_ref):
        o_ref[...] = jnp.dot(a_ref[...], b_ref[...],
                             preferred_element_type=jnp.float32)

    return pl.pallas_call(
        mmk,
        grid=(M // BM,),
        in_specs=[pl.BlockSpec((BM, Kd), lambda i: (i, 0)),
                  pl.BlockSpec((Kd, Nd), lambda i: (0, 0))],
        out_specs=pl.BlockSpec((BM, Nd), lambda i: (i, 0)),
        out_shape=jax.ShapeDtypeStruct((M, Nd), jnp.float32),
    )(a, b)


def _mesh():
    return plsc.VectorSubcoreMesh(core_axis_name="c", subcore_axis_name="s")


def _attn_kernel(srcs_p, dsts_p, asrc128, adst128, btile):
    @functools.partial(
        pl.kernel,
        out_type=jax.ShapeDtypeStruct((NW, E_PW * 4), jnp.float32),
        mesh=_mesh(),
        compiler_params=_CP,
        scratch_types=[
            pltpu.VMEM((E_PW,), jnp.int32),
            pltpu.VMEM((E_PW,), jnp.int32),
            pltpu.VMEM((SROWS,), jnp.float32),
            pltpu.VMEM((2, K, 128), jnp.float32),
            pltpu.VMEM((2, K, 128), jnp.float32),
            pltpu.VMEM((E_PW * 4,), jnp.float32),
            pltpu.VMEM((16,), jnp.float32),
            pltpu.SemaphoreType.DMA((2, 2)),
        ],
    )
    def body(srcs_hbm, dsts_hbm, asrc_hbm, adst_hbm, b_hbm, w_hbm,
             srcs_w, dsts_w, s_ref, ar_ring, ad_ring, pbuf, bvec, sems):
        wid = lax.axis_index("s") * NC + lax.axis_index("c")
        node0 = wid * NPW
        iota = lax.iota(jnp.int32, 16)
        lane_h = iota & 3
        lane_e = iota >> 2
        z16 = jnp.zeros((16,), jnp.float32)
        zi16 = jnp.zeros((16,), jnp.int32)

        pltpu.sync_copy(srcs_hbm.at[wid], srcs_w)
        pltpu.sync_copy(dsts_hbm.at[wid], dsts_w)
        pltpu.sync_copy(b_hbm, bvec)
        b = bvec[...]

        def zs(i, c):
            s_ref[pl.ds(i * 16, 16)] = z16
            return c
        lax.fori_loop(0, SROWS // 16, zs, 0)

        def start(ck, slot):
            pltpu.async_copy(asrc_hbm.at[srcs_w.at[pl.ds(ck * K, K)]],
                             ar_ring.at[slot], sems.at[0, slot])
            pltpu.async_copy(adst_hbm.at[dsts_w.at[pl.ds(ck * K, K)]],
                             ad_ring.at[slot], sems.at[1, slot])

        def wait(ck, slot):
            pltpu.make_async_copy(asrc_hbm.at[srcs_w.at[pl.ds(ck * K, K)]],
                                  ar_ring.at[slot], sems.at[0, slot]).wait()
            pltpu.make_async_copy(adst_hbm.at[dsts_w.at[pl.ds(ck * K, K)]],
                                  ad_ring.at[slot], sems.at[1, slot]).wait()

        start(0, 0)

        def p1(ck, c):
            slot = ck & 1

            @pl.when(ck + 1 < NCK)
            def _():
                start(ck + 1, 1 - slot)
            wait(ck, slot)
            sl = zi16 + slot
            for j in range(K // 4):
                le = 4 * j + lane_e
                av = plsc.load_gather(ar_ring, [sl, le, lane_h])
                dv = plsc.load_gather(ad_ring, [sl, le, lane_h])
                alpha = av + dv
                alpha = jnp.maximum(alpha, 0.2 * alpha)
                p = jnp.exp(alpha - b)
                pbuf[pl.ds(ck * (K * 4) + 16 * j, 16)] = p
                dstv = plsc.load_gather(dsts_w, [ck * K + le])
                sidx = (dstv - node0) * 4 + lane_h
                for k in range(4):
                    plsc.addupdate_scatter(s_ref, [sidx], p,
                                           mask=lane_e == k)
            return c
        lax.fori_loop(0, NCK, p1, 0)

        def rec(i, c):
            v = s_ref[pl.ds(i * 16, 16)]
            s_ref[pl.ds(i * 16, 16)] = 1.0 / (v + 1e-16)
            return c
        lax.fori_loop(0, SROWS // 16, rec, 0)

        def p2(ck, c):
            for j in range(K // 4):
                off = ck * (K * 4) + 16 * j
                le = 4 * j + lane_e
                dstv = plsc.load_gather(dsts_w, [ck * K + le])
                r = plsc.load_gather(s_ref, [(dstv - node0) * 4 + lane_h])
                pbuf[pl.ds(off, 16)] = pbuf[pl.ds(off, 16)] * r
            return c
        lax.fori_loop(0, NCK, p2, 0)
        pltpu.sync_copy(pbuf, w_hbm.at[wid])

    return body(srcs_p, dsts_p, asrc128, adst128, btile)


def _agg_kernel(srcs_p, dsts_p, w_p, h_pad):
    @functools.partial(
        pl.kernel,
        out_type=jax.ShapeDtypeStruct((4, N_PAD, HID), jnp.float32),
        mesh=_mesh(),
        compiler_params=_CP,
        scratch_types=[
            pltpu.VMEM((E_PW,), jnp.int32),
            pltpu.VMEM((E_PW,), jnp.int32),
            pltpu.VMEM((2, K, HID), jnp.float32),
            pltpu.VMEM((2 * K * 4,), jnp.float32),
            pltpu.VMEM((NPW + 1, HID), jnp.float32),
            pltpu.SemaphoreType.DMA((2, 2)),
        ],
    )
    def body(srcs_hbm, dsts_hbm, w_hbm, h_hbm, g4_hbm,
             srcs_w, dsts_w, h_ring, w_ring, gstage, sems):
        wid = lax.axis_index("s") * NC + lax.axis_index("c")
        node0 = wid * NPW
        iota = lax.iota(jnp.int32, 16)
        zi16 = jnp.zeros((16,), jnp.int32)
        zf16 = jnp.zeros((16,), jnp.float32)
        cols = [h * 64 + i * 16 + iota for h in range(HEADS) for i in range(4)]

        pltpu.sync_copy(srcs_hbm.at[wid], srcs_w)
        pltpu.sync_copy(dsts_hbm.at[wid], dsts_w)

        def start(ck, slot):
            pltpu.async_copy(h_hbm.at[srcs_w.at[pl.ds(ck * K, K)]],
                             h_ring.at[slot], sems.at[0, slot])
            pltpu.async_copy(w_hbm.at[wid, pl.ds(ck * (K * 4), K * 4)],
                             w_ring.at[pl.ds(slot * (K * 4), K * 4)],
                             sems.at[1, slot])

        def wait(ck, slot):
            pltpu.make_async_copy(h_hbm.at[srcs_w.at[pl.ds(ck * K, K)]],
                                  h_ring.at[slot], sems.at[0, slot]).wait()
            pltpu.make_async_copy(w_hbm.at[wid, pl.ds(ck * (K * 4), K * 4)],
                                  w_ring.at[pl.ds(slot * (K * 4), K * 4)],
                                  sems.at[1, slot]).wait()

        def flush_to(row_node, accs):
            row = zi16 + row_node
            for j in range(16):
                plsc.store_scatter(gstage, [row, cols[j]], accs[j])

        def q_body(q, c):
            hq_idx = [q * 64 + i * 16 + iota for i in range(4)]
            start(0, 0)

            def p2(ck, carry):
                slot = ck & 1

                @pl.when(ck + 1 < NCK)
                def _():
                    start(ck + 1, 1 - slot)
                wait(ck, slot)
                sl = zi16 + slot
                prev = carry[0]
                accs = list(carry[1:])
                d16 = zi16
                w16 = zf16
                for e in range(K):
                    if e % 16 == 0:
                        d16 = dsts_w[pl.ds(ck * K + e, 16)]
                    if e % 4 == 0:
                        w16 = w_ring[pl.ds(slot * (K * 4) + (e // 4) * 16, 16)]
                    dsc = d16[e % 16] - node0
                    changed = jnp.logical_and(dsc != prev, prev >= 0)

                    def fl(ops):
                        flush_to(ops[0], ops[1:])
                        return (zf16,) * 16

                    def kp(ops):
                        return tuple(ops[1:])

                    accs = list(lax.cond(changed, fl, kp, (prev, *accs)))
                    hqs = [plsc.load_gather(h_ring, [sl, zi16 + e, hq_idx[i]])
                           for i in range(4)]
                    for h in range(HEADS):
                        wh = w16[(e % 4) * 4 + h]
                        for i in range(4):
                            accs[h * 4 + i] = accs[h * 4 + i] + wh * hqs[i]
                    prev = dsc
                return (prev, *accs)

            carry = lax.fori_loop(0, NCK, p2,
                                  (jnp.int32(-1),) + (zf16,) * 16)

            @pl.when(carry[0] >= 0)
            def _():
                flush_to(carry[0], carry[1:])
            pltpu.sync_copy(gstage.at[pl.ds(0, NPW)],
                            g4_hbm.at[q, pl.ds(node0, NPW)])
            return c
        lax.fori_loop(0, 4, q_body, 0)

    return body(srcs_p, dsts_p, w_p, h_pad)


def kernel(x, edge_index, batch, t, cond, node_W, node_b, time_W1, time_b1,
           time_W2, time_b2, cond_W1, cond_b1, cond_W2, cond_b2,
           gat_W0, att_src0, att_dst0, gat_b0,
           gat_W1, att_src1, att_dst1, gat_b1,
           gat_W2, att_src2, att_dst2, gat_b2, out_W, out_b):
    n = x.shape[0]
    t_emb = jax.nn.relu(t[:, None] @ time_W1 + time_b1) @ time_W2 + time_b2
    c_emb = jax.nn.relu(cond @ cond_W1 + cond_b1) @ cond_W2 + cond_b2
    xp = jnp.pad(x, ((0, N_PAD - n), (0, 0)))
    emb = jnp.pad((t_emb + c_emb)[batch], ((0, N_PAD - n), (0, 0)))
    hp = _mm(xp, node_W) + node_b + emb

    loop = jnp.arange(n, dtype=edge_index.dtype)
    src = jnp.concatenate([edge_index[0], loop])
    dst = jnp.concatenate([edge_index[1], loop])
    order = jnp.argsort(dst)
    srcs, dsts = src[order], dst[order]
    bounds = jnp.arange(NW + 1, dtype=jnp.int32) * NPW
    estart = jnp.searchsorted(dsts, bounds).astype(jnp.int32)
    i = jnp.arange(E_PW, dtype=jnp.int32)[None, :]
    ne = (estart[1:] - estart[:-1])[:, None]
    idx = jnp.minimum(estart[:-1][:, None] + i, E - 1)
    valid = i < ne
    srcs_p = jnp.where(valid, srcs[idx], 0)
    dsts_p = jnp.where(valid, dsts[idx],
                       (jnp.arange(NW, dtype=jnp.int32) * NPW + NPW)[:, None])

    for (Wl, a_s, a_d, bl) in ((gat_W0, att_src0, att_dst0, gat_b0),
                               (gat_W1, att_src1, att_dst1, gat_b1),
                               (gat_W2, att_src2, att_dst2, gat_b2)):
        Wr = Wl.reshape(HID, HEADS, HID)
        A_src = jnp.einsum('chd,hd->ch', Wr, a_s)
        A_dst = jnp.einsum('chd,hd->ch', Wr, a_d)
        asrc128 = _mm(hp, jnp.pad(A_src, ((0, 0), (0, 124))))
        adst128 = jnp.pad(_mm(hp, jnp.pad(A_dst, ((0, 0), (0, 124)))),
                          ((0, 8), (0, 0)))
        b4 = jax.nn.leaky_relu(asrc128[:n, :4].max(0)
                               + adst128[:n, :4].max(0), 0.2)
        btile = jnp.tile(b4, 4)
        w_p = _attn_kernel(srcs_p, dsts_p, asrc128, adst128, btile)
        g4 = _agg_kernel(srcs_p, dsts_p, w_p, hp)
        G2 = (g4.reshape(4, N_PAD, HEADS, 64)
              .transpose(1, 2, 0, 3).reshape(N_PAD, HEADS * HID))
        Wcat = Wr.transpose(1, 0, 2).reshape(HEADS * HID, HID)
        hp = jax.nn.relu(_mm(G2, Wcat) / HEADS + bl)
    return _mm(hp, out_W)[:n] + out_b


# speedup vs baseline: 2.2167x; 1.8797x over previous
import functools

import jax
import jax.numpy as jnp
from jax import lax
from jax.experimental import pallas as pl
from jax.experimental.pallas import tpu as pltpu
from jax.experimental.pallas import tpu_sc as plsc

N = 10000
HID = 256
HEADS = 4
NC = 2
NS = 16
NW = NC * NS
NV = 128
NPV = 80
N_PAD = NV * NPV
NPW = N_PAD // NW
E = 320000 + N
E_PV = 3072
E_PW = 4 * E_PV
K = 16
NCK = E_PV // K
SROWS = (NPW + 8) * 4
AT_ROWS = (N_PAD + 8) * 4

_CP = pltpu.CompilerParams(needs_layout_passes=False)


def _mm(a, b):
    M, Kd = a.shape
    Nd = b.shape[1]
    BM = 512
    assert M % BM == 0

    def mmk(a_ref, b_ref, o_ref):
        o_ref[...] = jnp.dot(a_ref[...], b_ref[...],
                             preferred_element_type=jnp.float32)

    return pl.pallas_call(
        mmk,
        grid=(M // BM,),
        in_specs=[pl.BlockSpec((BM, Kd), lambda i: (i, 0)),
                  pl.BlockSpec((Kd, Nd), lambda i: (0, 0))],
        out_specs=pl.BlockSpec((BM, Nd), lambda i: (i, 0)),
        out_shape=jax.ShapeDtypeStruct((M, Nd), jnp.float32),
    )(a, b)


def _mesh():
    return plsc.VectorSubcoreMesh(core_axis_name="c", subcore_axis_name="s")


def _attn_kernel(srcs_f, dsts_f, asrc_t, adst_t, btile):
    @functools.partial(
        pl.kernel,
        out_type=jax.ShapeDtypeStruct((NV * E_PV * 4,), jnp.float32),
        mesh=_mesh(),
        compiler_params=_CP,
        scratch_types=[
            pltpu.VMEM((E_PW,), jnp.int32),
            pltpu.VMEM((E_PW,), jnp.int32),
            pltpu.VMEM((AT_ROWS,), jnp.float32),
            pltpu.VMEM((SROWS,), jnp.float32),
            pltpu.VMEM((SROWS,), jnp.float32),
            pltpu.VMEM((E_PW * 4,), jnp.float32),
            pltpu.VMEM((16,), jnp.float32),
        ],
    )
    def body(srcs_hbm, dsts_hbm, asrc_hbm, adst_hbm, b_hbm, w_hbm,
             srcs_w, dsts_w, asrc_w, adst_w, s_ref, pbuf, bvec):
        wid = lax.axis_index("s") * NC + lax.axis_index("c")
        node0 = wid * NPW
        iota = lax.iota(jnp.int32, 16)
        lane_h = iota & 3
        lane_e = iota >> 2
        z16 = jnp.zeros((16,), jnp.float32)

        pltpu.sync_copy(srcs_hbm.at[pl.ds(wid * E_PW, E_PW)], srcs_w)
        pltpu.sync_copy(dsts_hbm.at[pl.ds(wid * E_PW, E_PW)], dsts_w)
        pltpu.sync_copy(asrc_hbm, asrc_w)
        pltpu.sync_copy(adst_hbm.at[pl.ds(wid * (NPW * 4), SROWS)], adst_w)
        pltpu.sync_copy(b_hbm, bvec)
        b = bvec[...]

        def zs(i, c):
            s_ref[pl.ds(i * 16, 16)] = z16
            return c
        lax.fori_loop(0, SROWS // 16, zs, 0)

        def p1(j, c):
            le = 4 * j + lane_e
            srcv = plsc.load_gather(srcs_w, [le])
            dstv = plsc.load_gather(dsts_w, [le])
            av = plsc.load_gather(asrc_w, [srcv * 4 + lane_h])
            sidx = (dstv - node0) * 4 + lane_h
            dv = plsc.load_gather(adst_w, [sidx])
            alpha = av + dv
            alpha = jnp.maximum(alpha, 0.2 * alpha)
            p = jnp.exp(alpha - b)
            pbuf[pl.ds(j * 16, 16)] = p
            for k in range(4):
                plsc.addupdate_scatter(s_ref, [sidx], p, mask=lane_e == k)
            return c
        lax.fori_loop(0, E_PW // 4, p1, 0)

        def rec(i, c):
            v = s_ref[pl.ds(i * 16, 16)]
            s_ref[pl.ds(i * 16, 16)] = 1.0 / (v + 1e-16)
            return c
        lax.fori_loop(0, SROWS // 16, rec, 0)

        def p2(j, c):
            le = 4 * j + lane_e
            dstv = plsc.load_gather(dsts_w, [le])
            r = plsc.load_gather(s_ref, [(dstv - node0) * 4 + lane_h])
            pbuf[pl.ds(j * 16, 16)] = pbuf[pl.ds(j * 16, 16)] * r
            return c
        lax.fori_loop(0, E_PW // 4, p2, 0)
        pltpu.sync_copy(pbuf, w_hbm.at[pl.ds(wid * (E_PW * 4), E_PW * 4)])

    return body(srcs_f, dsts_f, asrc_t, adst_t, btile)


def _agg_kernel(srcs_f, dsts_f, w_f, h_pad, zg):
    @functools.partial(
        pl.kernel,
        out_type=jax.ShapeDtypeStruct((N_PAD, HEADS * HID), jnp.float32),
        mesh=_mesh(),
        compiler_params=_CP,
        scratch_types=[
            pltpu.VMEM((E_PV,), jnp.int32),
            pltpu.VMEM((E_PV,), jnp.int32),
            pltpu.VMEM((2, K, HID), jnp.float32),
            pltpu.VMEM((2 * K * 4,), jnp.float32),
            pltpu.VMEM((NPV + 1, HEADS * HID), jnp.float32),
            pltpu.SemaphoreType.DMA((2, 2)),
        ],
    )
    def body(srcs_hbm, dsts_hbm, w_hbm, h_hbm, zg_hbm, g_hbm,
             srcs_v, dsts_v, h_ring, w_ring, gstage, sems):
        wid = lax.axis_index("s") * NC + lax.axis_index("c")
        iota = lax.iota(jnp.int32, 16)
        zi16 = jnp.zeros((16,), jnp.int32)
        zf16 = jnp.zeros((16,), jnp.float32)
        cols = [h * HID + i * 16 + iota
                for h in range(HEADS) for i in range(16)]
        hq_idx = [i * 16 + iota for i in range(16)]

        def v_body(vl, c):
            v = wid * 4 + vl
            vnode0 = v * NPV
            pltpu.sync_copy(srcs_hbm.at[pl.ds(v * E_PV, E_PV)], srcs_v)
            pltpu.sync_copy(dsts_hbm.at[pl.ds(v * E_PV, E_PV)], dsts_v)
            pltpu.sync_copy(zg_hbm, gstage)

            def start(ck, slot):
                pltpu.async_copy(h_hbm.at[srcs_v.at[pl.ds(ck * K, K)]],
                                 h_ring.at[slot], sems.at[0, slot])
                pltpu.async_copy(
                    w_hbm.at[pl.ds(v * (E_PV * 4) + ck * (K * 4), K * 4)],
                    w_ring.at[pl.ds(slot * (K * 4), K * 4)],
                    sems.at[1, slot])

            def wait(ck, slot):
                pltpu.make_async_copy(h_hbm.at[srcs_v.at[pl.ds(ck * K, K)]],
                                      h_ring.at[slot],
                                      sems.at[0, slot]).wait()
                pltpu.make_async_copy(
                    w_hbm.at[pl.ds(v * (E_PV * 4) + ck * (K * 4), K * 4)],
                    w_ring.at[pl.ds(slot * (K * 4), K * 4)],
                    sems.at[1, slot]).wait()

            start(0, 0)

            def p2(ck, c2):
                slot = ck & 1

                @pl.when(ck + 1 < NCK)
                def _():
                    start(ck + 1, 1 - slot)
                wait(ck, slot)
                sl = zi16 + slot
                d16 = zi16
                w16 = zf16
                for e in range(K):
                    if e % 16 == 0:
                        d16 = dsts_v[pl.ds(ck * K + e, 16)]
                    if e % 4 == 0:
                        w16 = w_ring[pl.ds(slot * (K * 4) + (e // 4) * 16,
                                           16)]
                    rowv = zi16 + (d16[e % 16] - vnode0)
                    hqs = [plsc.load_gather(h_ring,
                                            [sl, zi16 + e, hq_idx[i]])
                           for i in range(16)]
                    for h in range(HEADS):
                        wh = w16[(e % 4) * 4 + h]
                        for i in range(16):
                            plsc.addupdate_scatter(
                                gstage, [rowv, cols[h * 16 + i]],
                                wh * hqs[i])
                return c2
            lax.fori_loop(0, NCK, p2, 0)
            pltpu.sync_copy(gstage.at[pl.ds(0, NPV)],
                            g_hbm.at[pl.ds(vnode0, NPV)])
            return c
        lax.fori_loop(0, 4, v_body, 0)

    return body(srcs_f, dsts_f, w_f, h_pad, zg)


def kernel(x, edge_index, batch, t, cond, node_W, node_b, time_W1, time_b1,
           time_W2, time_b2, cond_W1, cond_b1, cond_W2, cond_b2,
           gat_W0, att_src0, att_dst0, gat_b0,
           gat_W1, att_src1, att_dst1, gat_b1,
           gat_W2, att_src2, att_dst2, gat_b2, out_W, out_b):
    n = x.shape[0]
    t_emb = jax.nn.relu(t[:, None] @ time_W1 + time_b1) @ time_W2 + time_b2
    c_emb = jax.nn.relu(cond @ cond_W1 + cond_b1) @ cond_W2 + cond_b2
    xp = jnp.pad(x, ((0, N_PAD - n), (0, 0)))
    emb = jnp.pad((t_emb + c_emb)[batch], ((0, N_PAD - n), (0, 0)))
    hp = _mm(xp, node_W) + node_b + emb

    loop = jnp.arange(n, dtype=edge_index.dtype)
    src = jnp.concatenate([edge_index[0], loop])
    dst = jnp.concatenate([edge_index[1], loop])
    order = jnp.argsort(dst)
    srcs, dsts = src[order], dst[order]
    bounds = jnp.arange(NV + 1, dtype=jnp.int32) * NPV
    estart = jnp.searchsorted(dsts, bounds).astype(jnp.int32)
    i = jnp.arange(E_PV, dtype=jnp.int32)[None, :]
    ne = (estart[1:] - estart[:-1])[:, None]
    idx = jnp.minimum(estart[:-1][:, None] + i, E - 1)
    valid = i < ne
    srcs_f = jnp.where(valid, srcs[idx], N_PAD).reshape(-1)
    dsts_f = jnp.where(valid, dsts[idx],
                       (jnp.arange(NV, dtype=jnp.int32) * NPV + NPV)[:, None]
                       ).reshape(-1)

    zg = jnp.zeros((NPV + 1, HEADS * HID), jnp.float32)
    neg = jnp.full((8, 4), -1e30, jnp.float32)
    for (Wl, a_s, a_d, bl) in ((gat_W0, att_src0, att_dst0, gat_b0),
                               (gat_W1, att_src1, att_dst1, gat_b1),
                               (gat_W2, att_src2, att_dst2, gat_b2)):
        Wr = Wl.reshape(HID, HEADS, HID)
        A_src = jnp.einsum('chd,hd->ch', Wr, a_s)
        A_dst = jnp.einsum('chd,hd->ch', Wr, a_d)
        a_src = _mm(hp, jnp.pad(A_src, ((0, 0), (0, 124))))[:, :4]
        a_dst = _mm(hp, jnp.pad(A_dst, ((0, 0), (0, 124))))[:, :4]
        b4 = jax.nn.leaky_relu(a_src[:n].max(0) + a_dst[:n].max(0), 0.2)
        asrc_t = jnp.concatenate([a_src, neg]).reshape(-1)
        adst_t = jnp.pad(a_dst.reshape(-1), (0, 8 * 4))
        btile = jnp.tile(b4, 4)
        w_f = _attn_kernel(srcs_f, dsts_f, asrc_t, adst_t, btile)
        hp8 = jnp.pad(hp, ((0, 8), (0, 0)))
        g = _agg_kernel(srcs_f, dsts_f, w_f, hp8, zg)
        Wcat = Wr.transpose(1, 0, 2).reshape(HEADS * HID, HID)
        hp = jax.nn.relu(_mm(g, Wcat) / HEADS + bl)
    return _mm(hp, out_W)[:n] + out_b


# speedup vs baseline: 2.6760x; 1.2072x over previous
import functools

import jax
import jax.numpy as jnp
from jax import lax
from jax.experimental import pallas as pl
from jax.experimental.pallas import tpu as pltpu
from jax.experimental.pallas import tpu_sc as plsc

N = 10000
HID = 256
HEADS = 4
NC = 2
NS = 16
NW = NC * NS
NV = 128
NPV = 80
N_PAD = NV * NPV
NPW = N_PAD // NW
E = 320000 + N
E_PV = 3072
E_PW = 4 * E_PV
K = 16
NCK = E_PV // K
SROWS = (NPW + 8) * 4
AT_ROWS = (N_PAD + 8) * 4

_CP = pltpu.CompilerParams(needs_layout_passes=False)


def _mm(a, b):
    M, Kd = a.shape
    Nd = b.shape[1]
    BM = 512
    assert M % BM == 0

    def mmk(a_ref, b_ref, o_ref):
        o_ref[...] = jnp.dot(a_ref[...], b_ref[...],
                             preferred_element_type=jnp.float32)

    return pl.pallas_call(
        mmk,
        grid=(M // BM,),
        in_specs=[pl.BlockSpec((BM, Kd), lambda i: (i, 0)),
                  pl.BlockSpec((Kd, Nd), lambda i: (0, 0))],
        out_specs=pl.BlockSpec((BM, Nd), lambda i: (i, 0)),
        out_shape=jax.ShapeDtypeStruct((M, Nd), jnp.float32),
    )(a, b)


def _mesh():
    return plsc.VectorSubcoreMesh(core_axis_name="c", subcore_axis_name="s")


def _attn_kernel(srcs_f, dsts_f, asrc_t, adst_t, btile):
    @functools.partial(
        pl.kernel,
        out_type=jax.ShapeDtypeStruct((NV * E_PV * 4,), jnp.float32),
        mesh=_mesh(),
        compiler_params=_CP,
        scratch_types=[
            pltpu.VMEM((E_PW,), jnp.int32),
            pltpu.VMEM((E_PW,), jnp.int32),
            pltpu.VMEM((AT_ROWS,), jnp.float32),
            pltpu.VMEM((SROWS,), jnp.float32),
            pltpu.VMEM((SROWS,), jnp.float32),
            pltpu.VMEM((E_PW * 4,), jnp.float32),
            pltpu.VMEM((16,), jnp.float32),
        ],
    )
    def body(srcs_hbm, dsts_hbm, asrc_hbm, adst_hbm, b_hbm, w_hbm,
             srcs_w, dsts_w, asrc_w, adst_w, s_ref, pbuf, bvec):
        wid = lax.axis_index("s") * NC + lax.axis_index("c")
        node0 = wid * NPW
        iota = lax.iota(jnp.int32, 16)
        lane_h = iota & 3
        lane_e = iota >> 2
        z16 = jnp.zeros((16,), jnp.float32)

        pltpu.sync_copy(srcs_hbm.at[pl.ds(wid * E_PW, E_PW)], srcs_w)
        pltpu.sync_copy(dsts_hbm.at[pl.ds(wid * E_PW, E_PW)], dsts_w)
        pltpu.sync_copy(asrc_hbm, asrc_w)
        pltpu.sync_copy(adst_hbm.at[pl.ds(wid * (NPW * 4), SROWS)], adst_w)
        pltpu.sync_copy(b_hbm, bvec)
        b = bvec[...]

        def zs(i, c):
            s_ref[pl.ds(i * 16, 16)] = z16
            return c
        lax.fori_loop(0, SROWS // 16, zs, 0)

        def p1(j, c):
            le = 4 * j + lane_e
            srcv = plsc.load_gather(srcs_w, [le])
            dstv = plsc.load_gather(dsts_w, [le])
            av = plsc.load_gather(asrc_w, [srcv * 4 + lane_h])
            sidx = (dstv - node0) * 4 + lane_h
            dv = plsc.load_gather(adst_w, [sidx])
            alpha = av + dv
            alpha = jnp.maximum(alpha, 0.2 * alpha)
            p = jnp.exp(alpha - b)
            pbuf[pl.ds(j * 16, 16)] = p
            for k in range(4):
                plsc.addupdate_scatter(s_ref, [sidx], p, mask=lane_e == k)
            return c
        lax.fori_loop(0, E_PW // 4, p1, 0)

        def rec(i, c):
            v = s_ref[pl.ds(i * 16, 16)]
            s_ref[pl.ds(i * 16, 16)] = 1.0 / (v + 1e-16)
            return c
        lax.fori_loop(0, SROWS // 16, rec, 0)

        def p2(j, c):
            le = 4 * j + lane_e
            dstv = plsc.load_gather(dsts_w, [le])
            r = plsc.load_gather(s_ref, [(dstv - node0) * 4 + lane_h])
            pbuf[pl.ds(j * 16, 16)] = pbuf[pl.ds(j * 16, 16)] * r
            return c
        lax.fori_loop(0, E_PW // 4, p2, 0)
        pltpu.sync_copy(pbuf, w_hbm.at[pl.ds(wid * (E_PW * 4), E_PW * 4)])

    return body(srcs_f, dsts_f, asrc_t, adst_t, btile)


def _agg_kernel(srcs_f, dsts_f, w_f, h_pad, zg):
    @functools.partial(
        pl.kernel,
        out_type=jax.ShapeDtypeStruct((N_PAD, HEADS * HID), jnp.float32),
        mesh=_mesh(),
        compiler_params=_CP,
        scratch_types=[
            pltpu.VMEM((E_PV,), jnp.int32),
            pltpu.VMEM((E_PV,), jnp.int32),
            pltpu.VMEM((2, K, HID), jnp.float32),
            pltpu.VMEM((2 * K * 4,), jnp.float32),
            pltpu.VMEM((NPV + 1, HEADS * HID), jnp.float32),
            pltpu.SemaphoreType.DMA((2, 2)),
        ],
    )
    def body(srcs_hbm, dsts_hbm, w_hbm, h_hbm, zg_hbm, g_hbm,
             srcs_v, dsts_v, h_ring, w_ring, gstage, sems):
        wid = lax.axis_index("s") * NC + lax.axis_index("c")
        iota = lax.iota(jnp.int32, 16)
        zi16 = jnp.zeros((16,), jnp.int32)
        zf16 = jnp.zeros((16,), jnp.float32)
        cols = [h * HID + i * 16 + iota
                for h in range(HEADS) for i in range(16)]
        hq_idx = [i * 16 + iota for i in range(16)]

        def v_body(vl, c):
            v = wid * 4 + vl
            vnode0 = v * NPV
            pltpu.sync_copy(srcs_hbm.at[pl.ds(v * E_PV, E_PV)], srcs_v)
            pltpu.sync_copy(dsts_hbm.at[pl.ds(v * E_PV, E_PV)], dsts_v)
            pltpu.sync_copy(zg_hbm, gstage)

            def start(ck, slot):
                pltpu.async_copy(h_hbm.at[srcs_v.at[pl.ds(ck * K, K)]],
                                 h_ring.at[slot], sems.at[0, slot])
                pltpu.async_copy(
                    w_hbm.at[pl.ds(v * (E_PV * 4) + ck * (K * 4), K * 4)],
                    w_ring.at[pl.ds(slot * (K * 4), K * 4)],
                    sems.at[1, slot])

            def wait(ck, slot):
                pltpu.make_async_copy(h_hbm.at[srcs_v.at[pl.ds(ck * K, K)]],
                                      h_ring.at[slot],
                                      sems.at[0, slot]).wait()
                pltpu.make_async_copy(
                    w_hbm.at[pl.ds(v * (E_PV * 4) + ck * (K * 4), K * 4)],
                    w_ring.at[pl.ds(slot * (K * 4), K * 4)],
                    sems.at[1, slot]).wait()

            start(0, 0)

            def p2(ck, c2):
                slot = ck & 1

                @pl.when(ck + 1 < NCK)
                def _():
                    start(ck + 1, 1 - slot)
                wait(ck, slot)
                sl = zi16 + slot
                d16 = zi16
                w16 = zf16
                for e in range(K):
                    if e % 16 == 0:
                        d16 = dsts_v[pl.ds(ck * K + e, 16)]
                    if e % 4 == 0:
                        w16 = w_ring[pl.ds(slot * (K * 4) + (e // 4) * 16,
                                           16)]
                    rowv = zi16 + (d16[e % 16] - vnode0)
                    hqs = [plsc.load_gather(h_ring,
                                            [sl, zi16 + e, hq_idx[i]])
                           for i in range(16)]
                    for h in range(HEADS):
                        wh = w16[(e % 4) * 4 + h]
                        for i in range(16):
                            plsc.addupdate_scatter(
                                gstage, [rowv, cols[h * 16 + i]],
                                wh * hqs[i])
                return c2
            lax.fori_loop(0, NCK, p2, 0)
            pltpu.sync_copy(gstage.at[pl.ds(0, NPV)],
                            g_hbm.at[pl.ds(vnode0, NPV)])
            return c
        lax.fori_loop(0, 4, v_body, 0)

    return body(srcs_f, dsts_f, w_f, h_pad, zg)


def kernel(x, edge_index, batch, t, cond, node_W, node_b, time_W1, time_b1,
           time_W2, time_b2, cond_W1, cond_b1, cond_W2, cond_b2,
           gat_W0, att_src0, att_dst0, gat_b0,
           gat_W1, att_src1, att_dst1, gat_b1,
           gat_W2, att_src2, att_dst2, gat_b2, out_W, out_b):
    n = x.shape[0]
    t_emb = jax.nn.relu(t[:, None] @ time_W1 + time_b1) @ time_W2 + time_b2
    c_emb = jax.nn.relu(cond @ cond_W1 + cond_b1) @ cond_W2 + cond_b2
    xp = jnp.pad(x, ((0, N_PAD - n), (0, 0)))
    emb = jnp.pad((t_emb + c_emb)[batch], ((0, N_PAD - n), (0, 0)))
    hp = _mm(xp, node_W) + node_b + emb

    loop = jnp.arange(n, dtype=edge_index.dtype)
    src = jnp.concatenate([edge_index[0], loop])
    dst = jnp.concatenate([edge_index[1], loop])
    order = jnp.argsort(dst)
    srcs, dsts = src[order], dst[order]
    bounds = jnp.arange(NV + 1, dtype=jnp.int32) * NPV
    estart = jnp.searchsorted(dsts, bounds).astype(jnp.int32)
    i = jnp.arange(E_PV, dtype=jnp.int32)
    i = ((i % 8) * (E_PV // 8) + i // 8)[None, :]
    ne = (estart[1:] - estart[:-1])[:, None]
    idx = jnp.minimum(estart[:-1][:, None] + i, E - 1)
    valid = i < ne
    srcs_f = jnp.where(valid, srcs[idx], N_PAD).reshape(-1)
    dsts_f = jnp.where(valid, dsts[idx],
                       (jnp.arange(NV, dtype=jnp.int32) * NPV + NPV)[:, None]
                       ).reshape(-1)

    zg = jnp.zeros((NPV + 1, HEADS * HID), jnp.float32)
    neg = jnp.full((8, 4), -1e30, jnp.float32)
    for (Wl, a_s, a_d, bl) in ((gat_W0, att_src0, att_dst0, gat_b0),
                               (gat_W1, att_src1, att_dst1, gat_b1),
                               (gat_W2, att_src2, att_dst2, gat_b2)):
        Wr = Wl.reshape(HID, HEADS, HID)
        A_src = jnp.einsum('chd,hd->ch', Wr, a_s)
        A_dst = jnp.einsum('chd,hd->ch', Wr, a_d)
        a_src = _mm(hp, jnp.pad(A_src, ((0, 0), (0, 124))))[:, :4]
        a_dst = _mm(hp, jnp.pad(A_dst, ((0, 0), (0, 124))))[:, :4]
        b4 = jax.nn.leaky_relu(a_src[:n].max(0) + a_dst[:n].max(0), 0.2)
        asrc_t = jnp.concatenate([a_src, neg]).reshape(-1)
        adst_t = jnp.pad(a_dst.reshape(-1), (0, 8 * 4))
        btile = jnp.tile(b4, 4)
        w_f = _attn_kernel(srcs_f, dsts_f, asrc_t, adst_t, btile)
        hp8 = jnp.pad(hp, ((0, 8), (0, 0)))
        g = _agg_kernel(srcs_f, dsts_f, w_f, hp8, zg)
        Wcat = Wr.transpose(1, 0, 2).reshape(HEADS * HID, HID)
        hp = jax.nn.relu(_mm(g, Wcat) / HEADS + bl)
    return _mm(hp, out_W)[:n] + out_b
